# baseline (speedup 1.0000x reference)
import functools

import numpy as np

import jax
import jax.numpy as jnp
from jax import lax
from jax.experimental import pallas as pl
from jax.experimental.pallas import tpu as pltpu
from jax.experimental.pallas import tpu_sc as plsc

N = 50000
E = 800000
D = 64
DP = 72
NHALF = 25088
ROWS_PER_TILE = NHALF // 16
ACC_ROWS = NHALF + 16
DUMMY = NHALF + 8
CHUNK = 64
NCHG = E // CHUNK
CBASE = NCHG // 16
CREM = NCHG % 16
IRING = 8
RRING = 3
ROW_BLK = 1000

_PERM = np.zeros(D, dtype=np.int32)
for _q in range(2):
    for _j in range(16):
        _PERM[32 * _q + 2 * _j] = 32 * _q + _j
        _PERM[32 * _q + 2 * _j + 1] = 32 * _q + 16 + _j


def _ln_pad_body(x_ref, g_ref, b_ref, o_ref):
    x = x_ref[...]
    mu = jnp.mean(x, axis=1, keepdims=True)
    xc = x - mu
    var = jnp.mean(xc * xc, axis=1, keepdims=True)
    o_ref[...] = xc * lax.rsqrt(var + 1e-5) * g_ref[...] + b_ref[...]


def _ln_pad(h, g, b):
    return pl.pallas_call(
        _ln_pad_body,
        grid=(N // ROW_BLK,),
        in_specs=[
            pl.BlockSpec((ROW_BLK, D), lambda i: (i, 0)),
            pl.BlockSpec((1, D), lambda i: (0, 0)),
            pl.BlockSpec((1, D), lambda i: (0, 0)),
        ],
        out_specs=pl.BlockSpec((ROW_BLK, D), lambda i: (i, 0)),
        out_shape=jax.ShapeDtypeStruct((N, D), jnp.float32),
    )(h, g.reshape(1, D), b.reshape(1, D))


def _final_body(ap_ref, an_ref, wp_ref, wn_ref, c_ref, o_ref):
    ap = ap_ref[...]
    an = an_ref[...]
    xp = ap[:, :D] / jnp.maximum(ap[:, D:D + 1], 1.0)
    xn = an[:, :D] / jnp.maximum(an[:, D:D + 1], 1.0)
    y = (jnp.dot(xp, wp_ref[...], preferred_element_type=jnp.float32)
         + jnp.dot(xn, wn_ref[...], preferred_element_type=jnp.float32)
         + c_ref[...])
    o_ref[...] = jnp.clip(y, -50.0, 50.0)


def _final(aggp, aggn, wp, wn, c):
    return pl.pallas_call(
        _final_body,
        grid=(N // ROW_BLK,),
        in_specs=[
            pl.BlockSpec((ROW_BLK, DP), lambda i: (i, 0)),
            pl.BlockSpec((ROW_BLK, DP), lambda i: (i, 0)),
            pl.BlockSpec((D, D), lambda i: (0, 0)),
            pl.BlockSpec((D, D), lambda i: (0, 0)),
            pl.BlockSpec((1, D), lambda i: (0, 0)),
        ],
        out_specs=pl.BlockSpec((ROW_BLK, D), lambda i: (i, 0)),
        out_shape=jax.ShapeDtypeStruct((N, D), jnp.float32),
    )(aggp, aggn, wp, wn, c.reshape(1, D))


def _sc_body(hn_hbm, eip_hbm, ein_hbm, zeros_hbm, outp_hbm, outn_hbm,
             ei_buf, brow_bufs, rows_bufs, acc, gsem, ssem, isem):
    c = lax.axis_index("c")
    s = lax.axis_index("s")
    base = c * NHALF
    tile_row0 = s * ROWS_PER_TILE
    i32 = jnp.int32

    def drain_rows(sem):
        pltpu.make_async_copy(
            zeros_hbm.at[pl.ds(0, CHUNK)], rows_bufs.at[0], sem).wait()

    def drain_brow(sem):
        pltpu.make_async_copy(
            hn_hbm.at[pl.ds(0, CHUNK), :], brow_bufs.at[0], sem).wait()

    def drain_idx(sem):
        pltpu.make_async_copy(eip_hbm.at[0], ei_buf.at[0], sem).wait()

    def convert(p):
        hi_mask = i32(-65536)

        def cbody(r, carry):
            for q in range(2):
                xi = brow_bufs[p, r, pl.ds(q * 16, 16)]
                lo = lax.bitcast_convert_type(
                    lax.shift_left(xi, i32(16)), jnp.float32)
                hi = lax.bitcast_convert_type(xi & hi_mask, jnp.float32)
                rows_bufs[p, r, pl.ds(32 * q, 16)] = lo
                rows_bufs[p, r, pl.ds(32 * q + 16, 16)] = hi
            return carry
        lax.fori_loop(0, CHUNK, cbody, 0)

    cv = jnp.where(lax.iota(i32, 16) == 8, 1.0, 0.0).astype(jnp.float32)

    def ibody(r, carry):
        rows_bufs[0, r, pl.ds(56, 16)] = cv
        rows_bufs[1, r, pl.ds(56, 16)] = cv
        return carry
    lax.fori_loop(0, CHUNK, ibody, 0)

    def run_phase(ei_hbm, out_hbm):
        pltpu.sync_copy(zeros_hbm, acc.at[pl.ds(tile_row0, ROWS_PER_TILE)])
        plsc.subcore_barrier()

        count = jnp.where(s < CREM, CBASE + 1, CBASE)

        for jj in range(2):
            pltpu.async_copy(
                ei_hbm.at[s + 16 * jj], ei_buf.at[jj], isem.at[jj])

        def body(j, carry):
            slot = j % IRING
            p = j % 2

            @pl.when(j >= 2)
            def _():
                drain_rows(ssem.at[p])

            drain_idx(isem.at[slot])
            pltpu.async_copy(
                hn_hbm.at[ei_buf.at[slot, 0]], brow_bufs.at[p], gsem.at[p])

            for v in range(CHUNK // 16):
                d = ei_buf[slot, 1, pl.ds(v * 16, 16)] - base
                ok = (d >= 0) & (d < NHALF)
                ei_buf[slot, 1, pl.ds(v * 16, 16)] = jnp.where(ok, d, DUMMY)

            @pl.when(j + 2 < count)
            def _():
                slot2 = (j + 2) % IRING
                pltpu.async_copy(
                    ei_hbm.at[s + 16 * (j + 2)], ei_buf.at[slot2],
                    isem.at[slot2])

            @pl.when(j >= 1)
            def _():
                pj = (j - 1) % 2
                sj = (j - 1) % IRING
                drain_brow(gsem.at[pj])

                @pl.when(pj == 0)
                def _():
                    convert(0)

                @pl.when(pj == 1)
                def _():
                    convert(1)

                pltpu.async_copy(
                    rows_bufs.at[pj], acc.at[ei_buf.at[sj, 1]],
                    ssem.at[pj], add=True)
            return carry

        lax.fori_loop(0, count, body, 0)

        last = count - 1
        drain_brow(gsem.at[last % 2])

        @pl.when(last % 2 == 0)
        def _():
            convert(0)

        @pl.when(last % 2 == 1)
        def _():
            convert(1)

        pltpu.async_copy(
            rows_bufs.at[last % 2], acc.at[ei_buf.at[last % IRING, 1]],
            ssem.at[last % 2], add=True)
        drain_rows(ssem.at[last % 2])
        drain_rows(ssem.at[(last - 1) % 2])

        plsc.subcore_barrier()
        pltpu.sync_copy(
            acc.at[pl.ds(tile_row0, ROWS_PER_TILE)],
            out_hbm.at[pl.ds(base + tile_row0, ROWS_PER_TILE)])
        plsc.subcore_barrier()

    run_phase(eip_hbm, outp_hbm)
    run_phase(ein_hbm, outn_hbm)


@functools.partial(jax.jit, static_argnums=())
def _sc_segsum(hn, eip, ein, zeros):
    mesh = plsc.VectorSubcoreMesh(core_axis_name="c", subcore_axis_name="s")
    f = pl.kernel(
        _sc_body,
        mesh=mesh,
        compiler_params=pltpu.CompilerParams(use_tc_tiling_on_sc=False),
        out_type=[
            jax.ShapeDtypeStruct((2 * NHALF, DP), jnp.float32),
            jax.ShapeDtypeStruct((2 * NHALF, DP), jnp.float32),
        ],
        scratch_types=[
            pltpu.VMEM((IRING, 2, CHUNK), jnp.int32),
            pltpu.VMEM((2, CHUNK, D // 2), jnp.int32),
            pltpu.VMEM((2, CHUNK, DP), jnp.float32),
            pltpu.VMEM_SHARED((ACC_ROWS, DP), jnp.float32),
            pltpu.SemaphoreType.DMA((2,)),
            pltpu.SemaphoreType.DMA((2,)),
            pltpu.SemaphoreType.DMA((IRING,)),
        ],
    )
    return f(hn, eip, ein, zeros)


def kernel(t, h, edge_index_pos, edge_index_neg, ln_gamma, ln_beta,
           W_pos, b_pos, W_neg, b_neg, W_psi_pos, b_psi_pos,
           W_psi_neg, b_psi_neg):
    hn = _ln_pad(h, ln_gamma, ln_beta)
    hn_b = jnp.take(hn, _PERM, axis=1).astype(jnp.bfloat16)
    hn_packed = lax.bitcast_convert_type(
        hn_b.reshape(N, D // 2, 2), jnp.int32)
    zeros = jnp.zeros((ROWS_PER_TILE, DP), dtype=jnp.float32)
    eip = jnp.stack([edge_index_pos[0].reshape(NCHG, CHUNK),
                     edge_index_pos[1].reshape(NCHG, CHUNK)], axis=1)
    ein = jnp.stack([edge_index_neg[0].reshape(NCHG, CHUNK),
                     edge_index_neg[1].reshape(NCHG, CHUNK)], axis=1)
    aggp, aggn = _sc_segsum(hn_packed, eip, ein, zeros)
    wp = W_pos @ W_psi_pos
    wn = W_neg @ W_psi_neg
    cb = b_pos @ W_psi_pos + b_psi_pos + b_neg @ W_psi_neg + b_psi_neg
    return _final(aggp[:N], aggn[:N], wp, wn, cb)


# speedup vs baseline: 1.2656x; 1.2656x over previous
import functools

import jax
import jax.numpy as jnp
from jax import lax
from jax.experimental import pallas as pl
from jax.experimental.pallas import tpu as pltpu
from jax.experimental.pallas import tpu_sc as plsc

N = 50000
E = 800000
D = 64
DP = 72
NHALF = 25088
ROWS_PER_TILE = NHALF // 16
ACC_ROWS = NHALF + 16
DUMMY = NHALF + 8
CHUNK = 64
NCHG = E // CHUNK
CBASE = NCHG // 16
CREM = NCHG % 16
IRING = 8
RRING = 3
ROW_BLK = 1000


def _ln_pad_body(x_ref, g_ref, b_ref, o_ref):
    x = x_ref[...]
    mu = jnp.mean(x, axis=1, keepdims=True)
    xc = x - mu
    var = jnp.mean(xc * xc, axis=1, keepdims=True)
    y = xc * lax.rsqrt(var + 1e-5) * g_ref[...] + b_ref[...]
    col = lax.broadcasted_iota(jnp.int32, (ROW_BLK, DP - D), 1)
    pad = jnp.where(col == 0, 1.0, 0.0).astype(jnp.float32)
    o_ref[...] = jnp.concatenate([y, pad], axis=1)


def _ln_pad(h, g, b):
    return pl.pallas_call(
        _ln_pad_body,
        grid=(N // ROW_BLK,),
        in_specs=[
            pl.BlockSpec((ROW_BLK, D), lambda i: (i, 0)),
            pl.BlockSpec((1, D), lambda i: (0, 0)),
            pl.BlockSpec((1, D), lambda i: (0, 0)),
        ],
        out_specs=pl.BlockSpec((ROW_BLK, DP), lambda i: (i, 0)),
        out_shape=jax.ShapeDtypeStruct((N, DP), jnp.float32),
    )(h, g.reshape(1, D), b.reshape(1, D))


def _final_body(ap_ref, an_ref, wp_ref, wn_ref, c_ref, o_ref):
    ap = ap_ref[...]
    an = an_ref[...]
    xp = ap[:, :D] / jnp.maximum(ap[:, D:D + 1], 1.0)
    xn = an[:, :D] / jnp.maximum(an[:, D:D + 1], 1.0)
    y = (jnp.dot(xp, wp_ref[...], preferred_element_type=jnp.float32)
         + jnp.dot(xn, wn_ref[...], preferred_element_type=jnp.float32)
         + c_ref[...])
    o_ref[...] = jnp.clip(y, -50.0, 50.0)


def _final(aggp, aggn, wp, wn, c):
    return pl.pallas_call(
        _final_body,
        grid=(N // ROW_BLK,),
        in_specs=[
            pl.BlockSpec((ROW_BLK, DP), lambda i: (i, 0)),
            pl.BlockSpec((ROW_BLK, DP), lambda i: (i, 0)),
            pl.BlockSpec((D, D), lambda i: (0, 0)),
            pl.BlockSpec((D, D), lambda i: (0, 0)),
            pl.BlockSpec((1, D), lambda i: (0, 0)),
        ],
        out_specs=pl.BlockSpec((ROW_BLK, D), lambda i: (i, 0)),
        out_shape=jax.ShapeDtypeStruct((N, D), jnp.float32),
    )(aggp, aggn, wp, wn, c.reshape(1, D))


def _sc_body(hn_hbm, eip_hbm, ein_hbm, zeros_hbm, outp_hbm, outn_hbm,
             ei_buf, rows_bufs, acc, gsem, ssem, isem):
    c = lax.axis_index("c")
    s = lax.axis_index("s")
    base = c * NHALF
    tile_row0 = s * ROWS_PER_TILE
    i32 = jnp.int32

    def drain_rows(sem):
        pltpu.make_async_copy(
            zeros_hbm.at[pl.ds(0, CHUNK)], rows_bufs.at[0], sem).wait()

    def drain_idx(sem):
        pltpu.make_async_copy(eip_hbm.at[0], ei_buf.at[0], sem).wait()

    def run_phase(ei_hbm, out_hbm):
        pltpu.sync_copy(zeros_hbm, acc.at[pl.ds(tile_row0, ROWS_PER_TILE)])
        plsc.subcore_barrier()

        count = jnp.where(s < CREM, CBASE + 1, CBASE)

        for jj in range(2):
            pltpu.async_copy(
                ei_hbm.at[s + 16 * jj], ei_buf.at[jj], isem.at[jj])

        def body(j, carry):
            slot = j % IRING
            p = j % RRING

            @pl.when(j >= RRING)
            def _():
                drain_rows(ssem.at[p])

            drain_idx(isem.at[slot])
            pltpu.async_copy(
                hn_hbm.at[ei_buf.at[slot, 0]], rows_bufs.at[p], gsem.at[p])

            for v in range(CHUNK // 16):
                d = ei_buf[slot, 1, pl.ds(v * 16, 16)] - base
                ok = (d >= 0) & (d < NHALF)
                ei_buf[slot, 1, pl.ds(v * 16, 16)] = jnp.where(ok, d, DUMMY)

            @pl.when(j + 2 < count)
            def _():
                slot2 = (j + 2) % IRING
                pltpu.async_copy(
                    ei_hbm.at[s + 16 * (j + 2)], ei_buf.at[slot2],
                    isem.at[slot2])

            @pl.when(j >= 1)
            def _():
                pj = (j - 1) % RRING
                sj = (j - 1) % IRING
                drain_rows(gsem.at[pj])
                pltpu.async_copy(
                    rows_bufs.at[pj], acc.at[ei_buf.at[sj, 1]],
                    ssem.at[pj], add=True)
            return carry

        lax.fori_loop(0, count, body, 0)

        last = count - 1
        drain_rows(gsem.at[last % RRING])
        pltpu.async_copy(
            rows_bufs.at[last % RRING], acc.at[ei_buf.at[last % IRING, 1]],
            ssem.at[last % RRING], add=True)
        for q in range(RRING):
            drain_rows(ssem.at[(last - q) % RRING])

        plsc.subcore_barrier()
        pltpu.sync_copy(
            acc.at[pl.ds(tile_row0, ROWS_PER_TILE)],
            out_hbm.at[pl.ds(base + tile_row0, ROWS_PER_TILE)])
        plsc.subcore_barrier()

    run_phase(eip_hbm, outp_hbm)
    run_phase(ein_hbm, outn_hbm)


@functools.partial(jax.jit, static_argnums=())
def _sc_segsum(hn, eip, ein, zeros):
    mesh = plsc.VectorSubcoreMesh(core_axis_name="c", subcore_axis_name="s")
    f = pl.kernel(
        _sc_body,
        mesh=mesh,
        compiler_params=pltpu.CompilerParams(use_tc_tiling_on_sc=False),
        out_type=[
            jax.ShapeDtypeStruct((2 * NHALF, DP), jnp.float32),
            jax.ShapeDtypeStruct((2 * NHALF, DP), jnp.float32),
        ],
        scratch_types=[
            pltpu.VMEM((IRING, 2, CHUNK), jnp.int32),
            pltpu.VMEM((RRING, CHUNK, DP), jnp.float32),
            pltpu.VMEM_SHARED((ACC_ROWS, DP), jnp.float32),
            pltpu.SemaphoreType.DMA((RRING,)),
            pltpu.SemaphoreType.DMA((RRING,)),
            pltpu.SemaphoreType.DMA((IRING,)),
        ],
    )
    return f(hn, eip, ein, zeros)


def kernel(t, h, edge_index_pos, edge_index_neg, ln_gamma, ln_beta,
           W_pos, b_pos, W_neg, b_neg, W_psi_pos, b_psi_pos,
           W_psi_neg, b_psi_neg):
    hn = _ln_pad(h, ln_gamma, ln_beta)
    zeros = jnp.zeros((ROWS_PER_TILE, DP), dtype=jnp.float32)
    eip = jnp.stack([edge_index_pos[0].reshape(NCHG, CHUNK),
                     edge_index_pos[1].reshape(NCHG, CHUNK)], axis=1)
    ein = jnp.stack([edge_index_neg[0].reshape(NCHG, CHUNK),
                     edge_index_neg[1].reshape(NCHG, CHUNK)], axis=1)
    aggp, aggn = _sc_segsum(hn, eip, ein, zeros)
    wp = W_pos @ W_psi_pos
    wn = W_neg @ W_psi_neg
    cb = b_pos @ W_psi_pos + b_psi_pos + b_neg @ W_psi_neg + b_psi_neg
    return _final(aggp[:N], aggn[:N], wp, wn, cb)


# speedup vs baseline: 1.6005x; 1.2646x over previous
import functools

import jax
import jax.numpy as jnp
from jax import lax
from jax.experimental import pallas as pl
from jax.experimental.pallas import tpu as pltpu
from jax.experimental.pallas import tpu_sc as plsc

N = 50000
E = 800000
D = 64
DP = 72
NHALF = 25088
ROWS_PER_TILE = NHALF // 16
ACC_ROWS = NHALF + 16
DUMMY = NHALF + 8
CHUNK = 64
NCHG = E // CHUNK
CBASE = NCHG // 16
CREM = NCHG % 16
IRING = 8
RRING = 3
ROW_BLK = 1000


def _ln_pad_body(x_ref, g_ref, b_ref, o_ref):
    x = x_ref[...]
    mu = jnp.mean(x, axis=1, keepdims=True)
    xc = x - mu
    var = jnp.mean(xc * xc, axis=1, keepdims=True)
    y = xc * lax.rsqrt(var + 1e-5) * g_ref[...] + b_ref[...]
    col = lax.broadcasted_iota(jnp.int32, (ROW_BLK, DP - D), 1)
    pad = jnp.where(col == 0, 1.0, 0.0).astype(jnp.float32)
    o_ref[...] = jnp.concatenate([y, pad], axis=1)


def _ln_pad(h, g, b):
    return pl.pallas_call(
        _ln_pad_body,
        grid=(N // ROW_BLK,),
        in_specs=[
            pl.BlockSpec((ROW_BLK, D), lambda i: (i, 0)),
            pl.BlockSpec((1, D), lambda i: (0, 0)),
            pl.BlockSpec((1, D), lambda i: (0, 0)),
        ],
        out_specs=pl.BlockSpec((ROW_BLK, DP), lambda i: (i, 0)),
        out_shape=jax.ShapeDtypeStruct((N, DP), jnp.float32),
    )(h, g.reshape(1, D), b.reshape(1, D))


def _final_body(ap_ref, an_ref, wp_ref, wn_ref, c_ref, o_ref):
    ap = ap_ref[...]
    an = an_ref[...]
    xp = ap[:, :D] / jnp.maximum(ap[:, D:D + 1], 1.0)
    xn = an[:, :D] / jnp.maximum(an[:, D:D + 1], 1.0)
    y = (jnp.dot(xp, wp_ref[...], preferred_element_type=jnp.float32)
         + jnp.dot(xn, wn_ref[...], preferred_element_type=jnp.float32)
         + c_ref[...])
    o_ref[...] = jnp.clip(y, -50.0, 50.0)


def _final(aggp, aggn, wp, wn, c):
    return pl.pallas_call(
        _final_body,
        grid=(N // ROW_BLK,),
        in_specs=[
            pl.BlockSpec((ROW_BLK, DP), lambda i: (i, 0)),
            pl.BlockSpec((ROW_BLK, DP), lambda i: (i, 0)),
            pl.BlockSpec((D, D), lambda i: (0, 0)),
            pl.BlockSpec((D, D), lambda i: (0, 0)),
            pl.BlockSpec((1, D), lambda i: (0, 0)),
        ],
        out_specs=pl.BlockSpec((ROW_BLK, D), lambda i: (i, 0)),
        out_shape=jax.ShapeDtypeStruct((N, D), jnp.float32),
    )(aggp, aggn, wp, wn, c.reshape(1, D))


def _sc_body(hn_hbm, eip_hbm, ein_hbm, zeros_hbm, outp_hbm, outn_hbm,
             ei_buf, rows_bufs, acc, gsem, ssem, isem):
    c = lax.axis_index("c")
    s = lax.axis_index("s")
    base = c * NHALF
    tile_row0 = s * ROWS_PER_TILE
    i32 = jnp.int32

    def drain_rows(sem):
        pltpu.make_async_copy(
            zeros_hbm.at[pl.ds(0, CHUNK)], rows_bufs.at[0], sem).wait()

    def drain_idx(sem):
        pltpu.make_async_copy(eip_hbm.at[0], ei_buf.at[0], sem).wait()

    def run_phase(ei_hbm, out_hbm):
        pltpu.sync_copy(zeros_hbm, acc.at[pl.ds(tile_row0, ROWS_PER_TILE)])
        plsc.subcore_barrier()

        count = jnp.where(s < CREM, CBASE + 1, CBASE)

        for jj in range(2):
            pltpu.async_copy(
                ei_hbm.at[s + 16 * jj], ei_buf.at[jj], isem.at[jj])

        def body(j, carry):
            slot = j % IRING
            p = j % RRING

            @pl.when(j >= RRING)
            def _():
                drain_rows(ssem.at[p])

            drain_idx(isem.at[slot])
            pltpu.async_copy(
                hn_hbm.at[ei_buf.at[slot, 0]], rows_bufs.at[p], gsem.at[p])

            for v in range(CHUNK // 16):
                d = ei_buf[slot, 1, pl.ds(v * 16, 16)] - base
                ok = (d >= 0) & (d < NHALF)
                ei_buf[slot, 1, pl.ds(v * 16, 16)] = jnp.where(
                    ok, d, NHALF + s)

            @pl.when(j + 2 < count)
            def _():
                slot2 = (j + 2) % IRING
                pltpu.async_copy(
                    ei_hbm.at[s + 16 * (j + 2)], ei_buf.at[slot2],
                    isem.at[slot2])

            @pl.when(j >= 1)
            def _():
                pj = (j - 1) % RRING
                sj = (j - 1) % IRING
                drain_rows(gsem.at[pj])
                pltpu.async_copy(
                    rows_bufs.at[pj], acc.at[ei_buf.at[sj, 1]],
                    ssem.at[pj], add=True)
            return carry

        lax.fori_loop(0, count, body, 0)

        last = count - 1
        drain_rows(gsem.at[last % RRING])
        pltpu.async_copy(
            rows_bufs.at[last % RRING], acc.at[ei_buf.at[last % IRING, 1]],
            ssem.at[last % RRING], add=True)
        for q in range(RRING):
            drain_rows(ssem.at[(last - q) % RRING])

        plsc.subcore_barrier()
        pltpu.sync_copy(
            acc.at[pl.ds(tile_row0, ROWS_PER_TILE)],
            out_hbm.at[pl.ds(base + tile_row0, ROWS_PER_TILE)])
        plsc.subcore_barrier()

    run_phase(eip_hbm, outp_hbm)
    run_phase(ein_hbm, outn_hbm)


@functools.partial(jax.jit, static_argnums=())
def _sc_segsum(hn, eip, ein, zeros):
    mesh = plsc.VectorSubcoreMesh(core_axis_name="c", subcore_axis_name="s")
    f = pl.kernel(
        _sc_body,
        mesh=mesh,
        compiler_params=pltpu.CompilerParams(use_tc_tiling_on_sc=False),
        out_type=[
            jax.ShapeDtypeStruct((2 * NHALF, DP), jnp.float32),
            jax.ShapeDtypeStruct((2 * NHALF, DP), jnp.float32),
        ],
        scratch_types=[
            pltpu.VMEM((IRING, 2, CHUNK), jnp.int32),
            pltpu.VMEM((RRING, CHUNK, DP), jnp.float32),
            pltpu.VMEM_SHARED((ACC_ROWS, DP), jnp.float32),
            pltpu.SemaphoreType.DMA((RRING,)),
            pltpu.SemaphoreType.DMA((RRING,)),
            pltpu.SemaphoreType.DMA((IRING,)),
        ],
    )
    return f(hn, eip, ein, zeros)


def kernel(t, h, edge_index_pos, edge_index_neg, ln_gamma, ln_beta,
           W_pos, b_pos, W_neg, b_neg, W_psi_pos, b_psi_pos,
           W_psi_neg, b_psi_neg):
    hn = _ln_pad(h, ln_gamma, ln_beta)
    zeros = jnp.zeros((ROWS_PER_TILE, DP), dtype=jnp.float32)
    eip = jnp.stack([edge_index_pos[0].reshape(NCHG, CHUNK),
                     edge_index_pos[1].reshape(NCHG, CHUNK)], axis=1)
    ein = jnp.stack([edge_index_neg[0].reshape(NCHG, CHUNK),
                     edge_index_neg[1].reshape(NCHG, CHUNK)], axis=1)
    aggp, aggn = _sc_segsum(hn, eip, ein, zeros)
    wp = W_pos @ W_psi_pos
    wn = W_neg @ W_psi_neg
    cb = b_pos @ W_psi_pos + b_psi_pos + b_neg @ W_psi_neg + b_psi_neg
    return _final(aggp[:N], aggn[:N], wp, wn, cb)


# speedup vs baseline: 1.6007x; 1.0001x over previous
import functools

import jax
import jax.numpy as jnp
from jax import lax
from jax.experimental import pallas as pl
from jax.experimental.pallas import tpu as pltpu
from jax.experimental.pallas import tpu_sc as plsc

N = 50000
E = 800000
D = 64
DP = 72
NHALF = 25088
ROWS_PER_TILE = NHALF // 16
ACC_ROWS = NHALF + 256
DUMMY = NHALF + 8
CHUNK = 64
NCHG = E // CHUNK
CBASE = NCHG // 16
CREM = NCHG % 16
IRING = 8
RRING = 3
ROW_BLK = 1000


def _ln_pad_body(x_ref, g_ref, b_ref, o_ref):
    x = x_ref[...]
    mu = jnp.mean(x, axis=1, keepdims=True)
    xc = x - mu
    var = jnp.mean(xc * xc, axis=1, keepdims=True)
    y = xc * lax.rsqrt(var + 1e-5) * g_ref[...] + b_ref[...]
    col = lax.broadcasted_iota(jnp.int32, (ROW_BLK, DP - D), 1)
    pad = jnp.where(col == 0, 1.0, 0.0).astype(jnp.float32)
    o_ref[...] = jnp.concatenate([y, pad], axis=1)


def _ln_pad(h, g, b):
    return pl.pallas_call(
        _ln_pad_body,
        grid=(N // ROW_BLK,),
        in_specs=[
            pl.BlockSpec((ROW_BLK, D), lambda i: (i, 0)),
            pl.BlockSpec((1, D), lambda i: (0, 0)),
            pl.BlockSpec((1, D), lambda i: (0, 0)),
        ],
        out_specs=pl.BlockSpec((ROW_BLK, DP), lambda i: (i, 0)),
        out_shape=jax.ShapeDtypeStruct((N, DP), jnp.float32),
    )(h, g.reshape(1, D), b.reshape(1, D))


def _final_body(ap_ref, an_ref, wp_ref, wn_ref, c_ref, o_ref):
    ap = ap_ref[...]
    an = an_ref[...]
    xp = ap[:, :D] / jnp.maximum(ap[:, D:D + 1], 1.0)
    xn = an[:, :D] / jnp.maximum(an[:, D:D + 1], 1.0)
    y = (jnp.dot(xp, wp_ref[...], preferred_element_type=jnp.float32)
         + jnp.dot(xn, wn_ref[...], preferred_element_type=jnp.float32)
         + c_ref[...])
    o_ref[...] = jnp.clip(y, -50.0, 50.0)


def _final(aggp, aggn, wp, wn, c):
    return pl.pallas_call(
        _final_body,
        grid=(N // ROW_BLK,),
        in_specs=[
            pl.BlockSpec((ROW_BLK, DP), lambda i: (i, 0)),
            pl.BlockSpec((ROW_BLK, DP), lambda i: (i, 0)),
            pl.BlockSpec((D, D), lambda i: (0, 0)),
            pl.BlockSpec((D, D), lambda i: (0, 0)),
            pl.BlockSpec((1, D), lambda i: (0, 0)),
        ],
        out_specs=pl.BlockSpec((ROW_BLK, D), lambda i: (i, 0)),
        out_shape=jax.ShapeDtypeStruct((N, D), jnp.float32),
    )(aggp, aggn, wp, wn, c.reshape(1, D))


def _sc_body(hn_hbm, eip_hbm, ein_hbm, zeros_hbm, outp_hbm, outn_hbm,
             ei_buf, rows_bufs, acc, gsem, ssem, isem):
    c = lax.axis_index("c")
    s = lax.axis_index("s")
    base = c * NHALF
    tile_row0 = s * ROWS_PER_TILE
    i32 = jnp.int32

    def drain_rows(sem):
        pltpu.make_async_copy(
            zeros_hbm.at[pl.ds(0, CHUNK)], rows_bufs.at[0], sem).wait()

    def drain_idx(sem):
        pltpu.make_async_copy(eip_hbm.at[0], ei_buf.at[0], sem).wait()

    def run_phase(ei_hbm, out_hbm):
        pltpu.sync_copy(zeros_hbm, acc.at[pl.ds(tile_row0, ROWS_PER_TILE)])
        plsc.subcore_barrier()

        count = jnp.where(s < CREM, CBASE + 1, CBASE)

        for jj in range(2):
            pltpu.async_copy(
                ei_hbm.at[s + 16 * jj], ei_buf.at[jj], isem.at[jj])

        def body(j, carry):
            slot = j % IRING
            p = j % RRING

            @pl.when(j >= RRING)
            def _():
                drain_rows(ssem.at[p])

            drain_idx(isem.at[slot])
            pltpu.async_copy(
                hn_hbm.at[ei_buf.at[slot, 0]], rows_bufs.at[p], gsem.at[p])

            dummy = NHALF + s * 16 + lax.iota(i32, 16)
            for v in range(CHUNK // 16):
                d = ei_buf[slot, 1, pl.ds(v * 16, 16)] - base
                ok = (d >= 0) & (d < NHALF)
                ei_buf[slot, 1, pl.ds(v * 16, 16)] = jnp.where(ok, d, dummy)

            @pl.when(j + 2 < count)
            def _():
                slot2 = (j + 2) % IRING
                pltpu.async_copy(
                    ei_hbm.at[s + 16 * (j + 2)], ei_buf.at[slot2],
                    isem.at[slot2])

            @pl.when(j >= 1)
            def _():
                pj = (j - 1) % RRING
                sj = (j - 1) % IRING
                drain_rows(gsem.at[pj])
                pltpu.async_copy(
                    rows_bufs.at[pj], acc.at[ei_buf.at[sj, 1]],
                    ssem.at[pj], add=True)
            return carry

        lax.fori_loop(0, count, body, 0)

        last = count - 1
        drain_rows(gsem.at[last % RRING])
        pltpu.async_copy(
            rows_bufs.at[last % RRING], acc.at[ei_buf.at[last % IRING, 1]],
            ssem.at[last % RRING], add=True)
        for q in range(RRING):
            drain_rows(ssem.at[(last - q) % RRING])

        plsc.subcore_barrier()
        pltpu.sync_copy(
            acc.at[pl.ds(tile_row0, ROWS_PER_TILE)],
            out_hbm.at[pl.ds(base + tile_row0, ROWS_PER_TILE)])
        plsc.subcore_barrier()

    run_phase(eip_hbm, outp_hbm)
    run_phase(ein_hbm, outn_hbm)


@functools.partial(jax.jit, static_argnums=())
def _sc_segsum(hn, eip, ein, zeros):
    mesh = plsc.VectorSubcoreMesh(core_axis_name="c", subcore_axis_name="s")
    f = pl.kernel(
        _sc_body,
        mesh=mesh,
        compiler_params=pltpu.CompilerParams(use_tc_tiling_on_sc=False),
        out_type=[
            jax.ShapeDtypeStruct((2 * NHALF, DP), jnp.float32),
            jax.ShapeDtypeStruct((2 * NHALF, DP), jnp.float32),
        ],
        scratch_types=[
            pltpu.VMEM((IRING, 2, CHUNK), jnp.int32),
            pltpu.VMEM((RRING, CHUNK, DP), jnp.float32),
            pltpu.VMEM_SHARED((ACC_ROWS, DP), jnp.float32),
            pltpu.SemaphoreType.DMA((RRING,)),
            pltpu.SemaphoreType.DMA((RRING,)),
            pltpu.SemaphoreType.DMA((IRING,)),
        ],
    )
    return f(hn, eip, ein, zeros)


def kernel(t, h, edge_index_pos, edge_index_neg, ln_gamma, ln_beta,
           W_pos, b_pos, W_neg, b_neg, W_psi_pos, b_psi_pos,
           W_psi_neg, b_psi_neg):
    hn = _ln_pad(h, ln_gamma, ln_beta)
    zeros = jnp.zeros((ROWS_PER_TILE, DP), dtype=jnp.float32)
    eip = jnp.stack([edge_index_pos[0].reshape(NCHG, CHUNK),
                     edge_index_pos[1].reshape(NCHG, CHUNK)], axis=1)
    ein = jnp.stack([edge_index_neg[0].reshape(NCHG, CHUNK),
                     edge_index_neg[1].reshape(NCHG, CHUNK)], axis=1)
    aggp, aggn = _sc_segsum(hn, eip, ein, zeros)
    wp = W_pos @ W_psi_pos
    wn = W_neg @ W_psi_neg
    cb = b_pos @ W_psi_pos + b_psi_pos + b_neg @ W_psi_neg + b_psi_neg
    return _final(aggp[:N], aggn[:N], wp, wn, cb)


# speedup vs baseline: 1.7453x; 1.0903x over previous
import functools

import jax
import jax.numpy as jnp
from jax import lax
from jax.experimental import pallas as pl
from jax.experimental.pallas import tpu as pltpu
from jax.experimental.pallas import tpu_sc as plsc

N = 50000
E = 800000
D = 64
DP = 72
NHALF = 25088
ROWS_PER_TILE = NHALF // 16
ACC_ROWS = NHALF + 256
DUMMY = NHALF + 8
CHUNK = 64
NCHG = E // CHUNK
CBASE = NCHG // 16
CREM = NCHG % 16
IRING = 8
RRING = 3
ROW_BLK = 1000


def _ln_pad_body(x_ref, g_ref, b_ref, o_ref):
    x = x_ref[...]
    mu = jnp.mean(x, axis=1, keepdims=True)
    xc = x - mu
    var = jnp.mean(xc * xc, axis=1, keepdims=True)
    y = xc * lax.rsqrt(var + 1e-5) * g_ref[...] + b_ref[...]
    col = lax.broadcasted_iota(jnp.int32, (ROW_BLK, DP - D), 1)
    pad = jnp.where(col == 0, 1.0, 0.0).astype(jnp.float32)
    o_ref[...] = jnp.concatenate([y, pad], axis=1)


def _ln_pad(h, g, b):
    return pl.pallas_call(
        _ln_pad_body,
        grid=(N // ROW_BLK,),
        in_specs=[
            pl.BlockSpec((ROW_BLK, D), lambda i: (i, 0)),
            pl.BlockSpec((1, D), lambda i: (0, 0)),
            pl.BlockSpec((1, D), lambda i: (0, 0)),
        ],
        out_specs=pl.BlockSpec((ROW_BLK, DP), lambda i: (i, 0)),
        out_shape=jax.ShapeDtypeStruct((N, DP), jnp.float32),
    )(h, g.reshape(1, D), b.reshape(1, D))


def _final_body(ap_ref, an_ref, wp_ref, wn_ref, c_ref, o_ref):
    ap = ap_ref[...]
    an = an_ref[...]
    xp = ap[:, :D] / jnp.maximum(ap[:, D:D + 1], 1.0)
    xn = an[:, :D] / jnp.maximum(an[:, D:D + 1], 1.0)
    y = (jnp.dot(xp, wp_ref[...], preferred_element_type=jnp.float32)
         + jnp.dot(xn, wn_ref[...], preferred_element_type=jnp.float32)
         + c_ref[...])
    o_ref[...] = jnp.clip(y, -50.0, 50.0)


def _final(aggp, aggn, wp, wn, c):
    return pl.pallas_call(
        _final_body,
        grid=(N // ROW_BLK,),
        in_specs=[
            pl.BlockSpec((ROW_BLK, DP), lambda i: (i, 0)),
            pl.BlockSpec((ROW_BLK, DP), lambda i: (i, 0)),
            pl.BlockSpec((D, D), lambda i: (0, 0)),
            pl.BlockSpec((D, D), lambda i: (0, 0)),
            pl.BlockSpec((1, D), lambda i: (0, 0)),
        ],
        out_specs=pl.BlockSpec((ROW_BLK, D), lambda i: (i, 0)),
        out_shape=jax.ShapeDtypeStruct((N, D), jnp.float32),
    )(aggp, aggn, wp, wn, c.reshape(1, D))


def _sc_body(hn_hbm, eip_hbm, ein_hbm, zeros_hbm, outp_hbm, outn_hbm,
             ei_buf, rows_bufs, acc, gsem, ssem, isem):
    c = lax.axis_index("c")
    s = lax.axis_index("s")
    base = c * NHALF
    tile_row0 = s * ROWS_PER_TILE
    i32 = jnp.int32

    def drain_rows(sem):
        pltpu.make_async_copy(
            zeros_hbm.at[pl.ds(0, CHUNK)], rows_bufs.at[0], sem).wait()

    def drain_idx(sem):
        pltpu.make_async_copy(eip_hbm.at[0], ei_buf.at[0], sem).wait()

    def run_phase(ei_hbm, out_hbm):
        pltpu.sync_copy(zeros_hbm, acc.at[pl.ds(tile_row0, ROWS_PER_TILE)])
        plsc.subcore_barrier()

        count = jnp.where(s < CREM, CBASE + 1, CBASE)

        for jj in range(2):
            pltpu.async_copy(
                ei_hbm.at[s + 16 * jj], ei_buf.at[jj], isem.at[jj])

        def body(j, carry):
            slot = j % IRING
            p = j % RRING

            @pl.when(j >= RRING)
            def _():
                drain_rows(ssem.at[p])

            drain_idx(isem.at[slot])
            pltpu.async_copy(
                hn_hbm.at[ei_buf.at[slot, 0]], rows_bufs.at[p], gsem.at[p])

            dummy = NHALF + s * 16 + lax.iota(i32, 16)
            for v in range(CHUNK // 16):
                d = ei_buf[slot, 1, pl.ds(v * 16, 16)] - base
                ok = (d >= 0) & (d < NHALF)
                ei_buf[slot, 1, pl.ds(v * 16, 16)] = jnp.where(ok, d, dummy)

            @pl.when(j + 2 < count)
            def _():
                slot2 = (j + 2) % IRING
                pltpu.async_copy(
                    ei_hbm.at[s + 16 * (j + 2)], ei_buf.at[slot2],
                    isem.at[slot2])

            @pl.when(j >= 2)
            def _():
                pj = (j - 2) % RRING
                sj = (j - 2) % IRING
                drain_rows(gsem.at[pj])
                pltpu.async_copy(
                    rows_bufs.at[pj], acc.at[ei_buf.at[sj, 1]],
                    ssem.at[pj], add=True)
            return carry

        lax.fori_loop(0, count, body, 0)

        last = count - 1
        for off in (1, 0):
            kk = last - off
            drain_rows(gsem.at[kk % RRING])
            pltpu.async_copy(
                rows_bufs.at[kk % RRING], acc.at[ei_buf.at[kk % IRING, 1]],
                ssem.at[kk % RRING], add=True)
        for q in range(RRING):
            drain_rows(ssem.at[(last - q) % RRING])

        plsc.subcore_barrier()
        pltpu.sync_copy(
            acc.at[pl.ds(tile_row0, ROWS_PER_TILE)],
            out_hbm.at[pl.ds(base + tile_row0, ROWS_PER_TILE)])
        plsc.subcore_barrier()

    run_phase(eip_hbm, outp_hbm)
    run_phase(ein_hbm, outn_hbm)


@functools.partial(jax.jit, static_argnums=())
def _sc_segsum(hn, eip, ein, zeros):
    mesh = plsc.VectorSubcoreMesh(core_axis_name="c", subcore_axis_name="s")
    f = pl.kernel(
        _sc_body,
        mesh=mesh,
        compiler_params=pltpu.CompilerParams(use_tc_tiling_on_sc=False),
        out_type=[
            jax.ShapeDtypeStruct((2 * NHALF, DP), jnp.float32),
            jax.ShapeDtypeStruct((2 * NHALF, DP), jnp.float32),
        ],
        scratch_types=[
            pltpu.VMEM((IRING, 2, CHUNK), jnp.int32),
            pltpu.VMEM((RRING, CHUNK, DP), jnp.float32),
            pltpu.VMEM_SHARED((ACC_ROWS, DP), jnp.float32),
            pltpu.SemaphoreType.DMA((RRING,)),
            pltpu.SemaphoreType.DMA((RRING,)),
            pltpu.SemaphoreType.DMA((IRING,)),
        ],
    )
    return f(hn, eip, ein, zeros)


def kernel(t, h, edge_index_pos, edge_index_neg, ln_gamma, ln_beta,
           W_pos, b_pos, W_neg, b_neg, W_psi_pos, b_psi_pos,
           W_psi_neg, b_psi_neg):
    hn = _ln_pad(h, ln_gamma, ln_beta)
    zeros = jnp.zeros((ROWS_PER_TILE, DP), dtype=jnp.float32)
    eip = jnp.stack([edge_index_pos[0].reshape(NCHG, CHUNK),
                     edge_index_pos[1].reshape(NCHG, CHUNK)], axis=1)
    ein = jnp.stack([edge_index_neg[0].reshape(NCHG, CHUNK),
                     edge_index_neg[1].reshape(NCHG, CHUNK)], axis=1)
    aggp, aggn = _sc_segsum(hn, eip, ein, zeros)
    wp = W_pos @ W_psi_pos
    wn = W_neg @ W_psi_neg
    cb = b_pos @ W_psi_pos + b_psi_pos + b_neg @ W_psi_neg + b_psi_neg
    return _final(aggp[:N], aggn[:N], wp, wn, cb)


# speedup vs baseline: 1.7460x; 1.0004x over previous
import functools

import jax
import jax.numpy as jnp
from jax import lax
from jax.experimental import pallas as pl
from jax.experimental.pallas import tpu as pltpu
from jax.experimental.pallas import tpu_sc as plsc

N = 50000
E = 800000
D = 64
DP = 72
NHALF = 25088
ROWS_PER_TILE = NHALF // 16
ACC_ROWS = NHALF + 256
CHUNK = 64
NCHG = E // CHUNK
CBASE = NCHG // 16
CREM = NCHG % 16
IRING = 8
RRING = 3
ROW_BLK = 1000


def _ln_pad_body(x_ref, g_ref, b_ref, o_ref):
    x = x_ref[...]
    mu = jnp.mean(x, axis=1, keepdims=True)
    xc = x - mu
    var = jnp.mean(xc * xc, axis=1, keepdims=True)
    y = xc * lax.rsqrt(var + 1e-5) * g_ref[...] + b_ref[...]
    col = lax.broadcasted_iota(jnp.int32, (ROW_BLK, DP - D), 1)
    pad = jnp.where(col == 0, 1.0, 0.0).astype(jnp.float32)
    o_ref[...] = jnp.concatenate([y, pad], axis=1)


def _ln_pad(h, g, b):
    return pl.pallas_call(
        _ln_pad_body,
        grid=(N // ROW_BLK,),
        in_specs=[
            pl.BlockSpec((ROW_BLK, D), lambda i: (i, 0)),
            pl.BlockSpec((1, D), lambda i: (0, 0)),
            pl.BlockSpec((1, D), lambda i: (0, 0)),
        ],
        out_specs=pl.BlockSpec((ROW_BLK, DP), lambda i: (i, 0)),
        out_shape=jax.ShapeDtypeStruct((N, DP), jnp.float32),
    )(h, g.reshape(1, D), b.reshape(1, D))


def _final_body(ap_ref, an_ref, wp_ref, wn_ref, c_ref, o_ref):
    ap = ap_ref[...]
    an = an_ref[...]
    xp = ap[:, :D] / jnp.maximum(ap[:, D:D + 1], 1.0)
    xn = an[:, :D] / jnp.maximum(an[:, D:D + 1], 1.0)
    y = (jnp.dot(xp, wp_ref[...], preferred_element_type=jnp.float32)
         + jnp.dot(xn, wn_ref[...], preferred_element_type=jnp.float32)
         + c_ref[...])
    o_ref[...] = jnp.clip(y, -50.0, 50.0)


def _final(aggp, aggn, wp, wn, c):
    return pl.pallas_call(
        _final_body,
        grid=(N // ROW_BLK,),
        in_specs=[
            pl.BlockSpec((ROW_BLK, DP), lambda i: (i, 0)),
            pl.BlockSpec((ROW_BLK, DP), lambda i: (i, 0)),
            pl.BlockSpec((D, D), lambda i: (0, 0)),
            pl.BlockSpec((D, D), lambda i: (0, 0)),
            pl.BlockSpec((1, D), lambda i: (0, 0)),
        ],
        out_specs=pl.BlockSpec((ROW_BLK, D), lambda i: (i, 0)),
        out_shape=jax.ShapeDtypeStruct((N, D), jnp.float32),
    )(aggp, aggn, wp, wn, c.reshape(1, D))


def _sc_body(hn_hbm, eip_hbm, ein_hbm, zeros_hbm, outp_hbm, outn_hbm,
             ei_buf, rows_bufs, acc, gsem, ssem, isem):
    c = lax.axis_index("c")
    s = lax.axis_index("s")
    base = c * NHALF
    tile_row0 = s * ROWS_PER_TILE
    i32 = jnp.int32

    def drain_rows(sem):
        pltpu.make_async_copy(
            zeros_hbm.at[pl.ds(0, CHUNK)], rows_bufs.at[0], sem).wait()

    def drain_idx(sem):
        pltpu.make_async_copy(eip_hbm.at[0], ei_buf.at[0], sem).wait()

    def run_phase(ei_hbm, out_hbm):
        pltpu.sync_copy(zeros_hbm, acc.at[pl.ds(tile_row0, ROWS_PER_TILE)])
        plsc.subcore_barrier()

        count = jnp.where(s < CREM, CBASE + 1, CBASE)

        for jj in range(2):
            pltpu.async_copy(
                ei_hbm.at[s + 16 * jj], ei_buf.at[jj], isem.at[jj])

        def body(j, carry):
            slot = j % IRING
            p = j % RRING

            @pl.when(j >= RRING)
            def _():
                drain_rows(ssem.at[p])

            drain_idx(isem.at[slot])
            pltpu.async_copy(
                hn_hbm.at[ei_buf.at[slot, 0]], rows_bufs.at[p], gsem.at[p])

            dummy = NHALF + s * 16 + lax.iota(i32, 16)
            for v in range(CHUNK // 16):
                d = ei_buf[slot, 1, pl.ds(v * 16, 16)] - base
                ok = (d >= 0) & (d < NHALF)
                ei_buf[slot, 1, pl.ds(v * 16, 16)] = jnp.where(ok, d, dummy)

            @pl.when(j + 2 < count)
            def _():
                slot2 = (j + 2) % IRING
                pltpu.async_copy(
                    ei_hbm.at[s + 16 * (j + 2)], ei_buf.at[slot2],
                    isem.at[slot2])

            @pl.when(j >= 2)
            def _():
                pj = (j - 2) % RRING
                sj = (j - 2) % IRING
                drain_rows(gsem.at[pj])
                pltpu.async_copy(
                    rows_bufs.at[pj], acc.at[ei_buf.at[sj, 1]],
                    ssem.at[pj], add=True)
            return carry

        lax.fori_loop(0, count, body, 0)

        last = count - 1
        for off in (1, 0):
            kk = last - off
            drain_rows(gsem.at[kk % RRING])
            pltpu.async_copy(
                rows_bufs.at[kk % RRING], acc.at[ei_buf.at[kk % IRING, 1]],
                ssem.at[kk % RRING], add=True)
        for q in range(RRING):
            drain_rows(ssem.at[(last - q) % RRING])

        plsc.subcore_barrier()
        pltpu.sync_copy(
            acc.at[pl.ds(tile_row0, ROWS_PER_TILE)],
            out_hbm.at[pl.ds(base + tile_row0, ROWS_PER_TILE)])
        plsc.subcore_barrier()

    run_phase(eip_hbm, outp_hbm)
    run_phase(ein_hbm, outn_hbm)


@functools.partial(jax.jit, static_argnums=())
def _sc_segsum(hn, eip, ein, zeros):
    mesh = plsc.VectorSubcoreMesh(core_axis_name="c", subcore_axis_name="s")
    f = pl.kernel(
        _sc_body,
        mesh=mesh,
        compiler_params=pltpu.CompilerParams(use_tc_tiling_on_sc=False),
        out_type=[
            jax.ShapeDtypeStruct((2 * NHALF, DP), jnp.float32),
            jax.ShapeDtypeStruct((2 * NHALF, DP), jnp.float32),
        ],
        scratch_types=[
            pltpu.VMEM((IRING, 2, CHUNK), jnp.int32),
            pltpu.VMEM((RRING, CHUNK, DP), jnp.float32),
            pltpu.VMEM_SHARED((ACC_ROWS, DP), jnp.float32),
            pltpu.SemaphoreType.DMA((RRING,)),
            pltpu.SemaphoreType.DMA((RRING,)),
            pltpu.SemaphoreType.DMA((IRING,)),
        ],
    )
    return f(hn, eip, ein, zeros)


def kernel(t, h, edge_index_pos, edge_index_neg, ln_gamma, ln_beta,
           W_pos, b_pos, W_neg, b_neg, W_psi_pos, b_psi_pos,
           W_psi_neg, b_psi_neg):
    hn = _ln_pad(h, ln_gamma, ln_beta)
    zeros = jnp.zeros((ROWS_PER_TILE, DP), dtype=jnp.float32)
    eip = jnp.stack([edge_index_pos[0].reshape(NCHG, CHUNK),
                     edge_index_pos[1].reshape(NCHG, CHUNK)], axis=1)
    ein = jnp.stack([edge_index_neg[0].reshape(NCHG, CHUNK),
                     edge_index_neg[1].reshape(NCHG, CHUNK)], axis=1)
    aggp, aggn = _sc_segsum(hn, eip, ein, zeros)
    wp = W_pos @ W_psi_pos
    wn = W_neg @ W_psi_neg
    cb = b_pos @ W_psi_pos + b_psi_pos + b_neg @ W_psi_neg + b_psi_neg
    return _final(aggp[:N], aggn[:N], wp, wn, cb)
